# baseline (speedup 1.0000x reference)
import functools

import jax
import jax.numpy as jnp
from jax import lax
from jax.experimental import pallas as pl
from jax.experimental.pallas import tpu as pltpu
from jax.experimental.pallas import tpu_sc as plsc

B = 4
Q = 2048
N = 8192
C = 256
S = 32
R2 = float(0.1 * 0.1)

NC = 2
NS = 16
NW = NC * NS

QPW = (B * Q) // NW
WPB = Q // QPW
NCHUNK = N // 16
QC = 512
CPW = C // NS

_mesh = plsc.VectorSubcoreMesh(core_axis_name="c", subcore_axis_name="s")
_cparams = pltpu.CompilerParams(needs_layout_passes=False)


def _fused_body(sup_hbm, q_hbm, feat_hbm, idx_hbm, gxyz_hbm, gfeat_hbm,
                supp_raw, supp_v, q_v, selbuf, idxout, xyzout,
                idx_c, feat_v, out_v):
    core = lax.axis_index("c")
    sub = lax.axis_index("s")
    wid = core * NS + sub
    b = wid // WPB
    qs = (wid % WPB) * QPW

    pltpu.sync_copy(sup_hbm.at[pl.ds(b * 3 * N, 3 * N)], supp_raw)
    pltpu.sync_copy(q_hbm.at[pl.ds((b * Q + qs) * 3, QPW * 3)], q_v)

    iota = jnp.arange(16, dtype=jnp.int32)
    zeros16 = jnp.zeros((16,), jnp.int32)
    t3 = iota * 3

    @plsc.parallel_loop(0, NCHUNK, unroll=8)
    def deinterleave(j):
        i3 = jnp.full((16,), j * 48, jnp.int32) + t3
        supp_v[pl.ds(j * 16, 16)] = plsc.load_gather(supp_raw, [i3])
        supp_v[pl.ds(N + j * 16, 16)] = plsc.load_gather(supp_raw, [i3 + 1])
        supp_v[pl.ds(2 * N + j * 16, 16)] = plsc.load_gather(supp_raw, [i3 + 2])

    def per_query(qq, carry):
        base3 = qq * 3
        qx = plsc.load_gather(q_v, [jnp.full((16,), base3, jnp.int32)])
        qy = plsc.load_gather(q_v, [jnp.full((16,), base3 + 1, jnp.int32)])
        qz = plsc.load_gather(q_v, [jnp.full((16,), base3 + 2, jnp.int32)])

        selbuf[pl.ds(0, 16)] = zeros16

        @plsc.parallel_loop(0, NCHUNK, unroll=8, carry=zeros16)
        def scan_chunks(j, cnt):
            xs = supp_v[pl.ds(j * 16, 16)]
            ys = supp_v[pl.ds(N + j * 16, 16)]
            zs = supp_v[pl.ds(2 * N + j * 16, 16)]
            dx = xs - qx
            dy = ys - qy
            dz = zs - qz
            d2 = (dx * dx + dy * dy) + dz * dz
            m = d2 < R2
            pre = plsc.cumsum(m.astype(jnp.int32))
            slot = cnt + pre - 1
            idxv = jnp.full((16,), j * 16, jnp.int32) + iota
            wm = jnp.logical_and(m, slot < 48)
            plsc.store_scatter(selbuf, [slot], idxv, mask=wm)
            return cnt + plsc.all_reduce_population_count(m)

        cnt = scan_chunks

        b0 = selbuf[pl.ds(0, 16)]
        b1 = selbuf[pl.ds(16, 16)]
        first = jnp.full((16,), jnp.max(jnp.where(iota < 1, b0, 0)), jnp.int32)
        id0 = jnp.where(iota < cnt, b0, first)
        id1 = jnp.where((iota + 16) < cnt, b1, first)

        idxout[pl.ds(qq * S, 16)] = id0
        idxout[pl.ds(qq * S + 16, 16)] = id1

        gx0 = plsc.load_gather(supp_v, [id0]) - qx
        gx1 = plsc.load_gather(supp_v, [id1]) - qx
        gy0 = plsc.load_gather(supp_v, [id0 + N]) - qy
        gy1 = plsc.load_gather(supp_v, [id1 + N]) - qy
        gz0 = plsc.load_gather(supp_v, [id0 + 2 * N]) - qz
        gz1 = plsc.load_gather(supp_v, [id1 + 2 * N]) - qz

        xyzout[pl.ds(qq * S, 16)] = gx0
        xyzout[pl.ds(qq * S + 16, 16)] = gx1
        xyzout[pl.ds(QPW * S + qq * S, 16)] = gy0
        xyzout[pl.ds(QPW * S + qq * S + 16, 16)] = gy1
        xyzout[pl.ds(2 * QPW * S + qq * S, 16)] = gz0
        xyzout[pl.ds(2 * QPW * S + qq * S + 16, 16)] = gz1
        return carry

    lax.fori_loop(0, QPW, per_query, jnp.int32(0))

    pltpu.sync_copy(idxout, idx_hbm.at[pl.ds((b * Q + qs) * S, QPW * S)])
    for d in range(3):
        pltpu.sync_copy(
            xyzout.at[pl.ds(d * QPW * S, QPW * S)],
            gxyz_hbm.at[pl.ds(((b * 3 + d) * Q + qs) * S, QPW * S)],
        )

    plsc.subcore_barrier()

    def per_lb(lb, carry):
        b2 = core * 2 + lb

        def per_qchunk(qc, carry2):
            pltpu.sync_copy(
                idx_hbm.at[pl.ds((b2 * Q + qc * QC) * S, QC * S)], idx_c)

            def per_chan(c, carry3):
                ch = sub * CPW + c
                pltpu.sync_copy(
                    feat_hbm.at[pl.ds((b2 * C + ch) * N, N)], feat_v)

                @plsc.parallel_loop(0, QC, unroll=16)
                def per_q(q):
                    i0 = idx_c[pl.ds(q * S, 16)]
                    i1 = idx_c[pl.ds(q * S + 16, 16)]
                    out_v[pl.ds(q * S, 16)] = plsc.load_gather(feat_v, [i0])
                    out_v[pl.ds(q * S + 16, 16)] = plsc.load_gather(
                        feat_v, [i1])

                pltpu.sync_copy(
                    out_v,
                    gfeat_hbm.at[pl.ds(((b2 * C + ch) * Q + qc * QC) * S,
                                       QC * S)],
                )
                return carry3

            lax.fori_loop(0, CPW, per_chan, jnp.int32(0))
            return carry2

        lax.fori_loop(0, Q // QC, per_qchunk, jnp.int32(0))
        return carry

    lax.fori_loop(0, 2, per_lb, jnp.int32(0))


_fused = functools.partial(
    pl.kernel,
    mesh=_mesh,
    compiler_params=_cparams,
    out_type=[
        jax.ShapeDtypeStruct((B * Q * S,), jnp.int32),
        jax.ShapeDtypeStruct((B * 3 * Q * S,), jnp.float32),
        jax.ShapeDtypeStruct((B * C * Q * S,), jnp.float32),
    ],
    scratch_types=[
        pltpu.VMEM((3 * N,), jnp.float32),
        pltpu.VMEM((3 * N,), jnp.float32),
        pltpu.VMEM((QPW * 3,), jnp.float32),
        pltpu.VMEM((48,), jnp.int32),
        pltpu.VMEM((QPW * S,), jnp.int32),
        pltpu.VMEM((3 * QPW * S,), jnp.float32),
        pltpu.VMEM((QC * S,), jnp.int32),
        pltpu.VMEM((N,), jnp.float32),
        pltpu.VMEM((QC * S,), jnp.float32),
    ],
)(_fused_body)


@jax.jit
def kernel(query_xyz, support_xyz, features):
    sup_flat = support_xyz.reshape(-1)
    q_flat = query_xyz.reshape(-1)
    _, gxyz_flat, gfeat_flat = _fused(sup_flat, q_flat, features.reshape(-1))
    grouped_xyz = gxyz_flat.reshape(B, 3, Q, S)
    grouped_features = gfeat_flat.reshape(B, C, Q, S)
    return grouped_xyz, grouped_features
